# baseline (device time: 72561 ns/iter reference)
import jax
import jax.numpy as jnp
from jax import lax
from jax.experimental import pallas as pl
from jax.experimental.pallas import tpu as pltpu

N_DEV = 4


def kernel(x, w_mat):
    m_total, k_per = x.shape
    k_total, n = w_mat.shape
    m_per = m_total // N_DEV

    def body(x_hbm, w_hbm, out_hbm, comm_ref, acc_ref, xstage_ref, xb_ref,
             commv_ref, wstage_ref, xsems, send_sems, recv_sems, wsems,
             csems, out_sems):
        my = lax.axis_index("i")

        m_h = m_per // 2

        def x_copy(row, slot, h):
            return pltpu.make_async_copy(
                x_hbm.at[pl.ds(row * m_per + h * m_h, m_h), :],
                xstage_ref.at[slot, pl.ds(h * m_h, m_h)],
                xsems.at[2 * slot + h],
            )

        xcs = []
        for d in range(1, N_DEV):
            row = lax.rem(my + d, N_DEV)
            halves = []
            for h in range(2):
                c = x_copy(row, d - 1, h)
                c.start()
                halves.append(c)
            xcs.append(halves)
        xc_own = []
        for h in range(2):
            c = x_copy(my, N_DEV - 1, h)
            c.start()
            xc_own.append(c)

        def w_copy(kb, slot):
            return pltpu.make_async_copy(
                w_hbm.at[pl.ds(kb * k_per, k_per), :],
                wstage_ref.at[slot],
                wsems.at[slot],
            )

        wc = [w_copy(my, 0)]
        wc[0].start()

        barrier = pltpu.get_barrier_semaphore()
        for d in range(1, N_DEV):
            peer = lax.rem(my + d, N_DEV)
            pl.semaphore_signal(
                barrier, inc=1,
                device_id=(peer,), device_id_type=pl.DeviceIdType.MESH,
            )
        pl.semaphore_wait(barrier, N_DEV - 1)

        rdmas = []
        for d in range(1, N_DEV):
            peer = lax.rem(my + d, N_DEV)
            halves = []
            for h in range(2):
                rs = pl.ds(h * m_h, m_h)
                xcs[d - 1][h].wait()
                xb_ref[d - 1, rs] = xstage_ref[d - 1][h * m_h:(h + 1) * m_h,
                                                      :].astype(jnp.bfloat16)
                rdma = pltpu.make_async_remote_copy(
                    src_ref=xb_ref.at[d - 1, rs],
                    dst_ref=comm_ref.at[d - 1, rs],
                    send_sem=send_sems.at[2 * (d - 1) + h],
                    recv_sem=recv_sems.at[2 * (d - 1) + h],
                    device_id=(peer,),
                    device_id_type=pl.DeviceIdType.MESH,
                )
                rdma.start()
                halves.append(rdma)
            rdmas.append(halves)

        wc.append(w_copy(lax.rem(my + N_DEV - 1, N_DEV), 1))
        wc[1].start()

        for c in xc_own:
            c.wait()
        wc[0].wait()
        acc_ref[...] = jnp.dot(
            xstage_ref[N_DEV - 1].astype(jnp.bfloat16),
            wstage_ref[0].astype(jnp.bfloat16),
            preferred_element_type=jnp.float32,
        )

        for d in range(1, N_DEV - 1):
            slot = d % 2
            kb_next = lax.rem(my - (d + 1) + N_DEV, N_DEV)
            nxt = w_copy(kb_next, (d + 1) % 2)
            nxt.start()
            wc.append(nxt)
            rdmas[d - 1][0].wait_recv()
            rdmas[d - 1][1].wait_recv()
            cc = pltpu.make_async_copy(
                comm_ref.at[d - 1], commv_ref.at[slot], csems.at[slot]
            )
            cc.start()
            cc.wait()
            wc[d].wait()
            acc_ref[...] += jnp.dot(
                commv_ref[slot],
                wstage_ref[slot].astype(jnp.bfloat16),
                preferred_element_type=jnp.float32,
            )

        d = N_DEV - 1
        rdmas[d - 1][0].wait_recv()
        rdmas[d - 1][1].wait_recv()
        cc = pltpu.make_async_copy(
            comm_ref.at[d - 1], commv_ref.at[d % 2], csems.at[d % 2]
        )
        cc.start()
        cc.wait()
        wc[d].wait()
        w_last = wstage_ref[d % 2].astype(jnp.bfloat16)
        n_chunks = 4
        m_c = m_per // n_chunks
        out_copies = []
        for c in range(n_chunks):
            rs = pl.ds(c * m_c, m_c)
            y = acc_ref[rs, :] + jnp.dot(
                commv_ref[d % 2][c * m_c:(c + 1) * m_c, :],
                w_last,
                preferred_element_type=jnp.float32,
            )
            acc_ref[rs, :] = y * jax.nn.sigmoid(y)
            oc = pltpu.make_async_copy(
                acc_ref.at[rs, :], out_hbm.at[rs, :], out_sems.at[c]
            )
            oc.start()
            out_copies.append(oc)

        for halves in rdmas:
            for r in halves:
                r.wait_send()
        for oc in out_copies:
            oc.wait()

    out, _ = pl.pallas_call(
        body,
        out_shape=(
            jax.ShapeDtypeStruct((m_per, n), jnp.float32),
            jax.ShapeDtypeStruct((N_DEV - 1, m_per, k_per), jnp.bfloat16),
        ),
        in_specs=[
            pl.BlockSpec(memory_space=pl.ANY),
            pl.BlockSpec(memory_space=pl.ANY),
        ],
        out_specs=(
            pl.BlockSpec(memory_space=pl.ANY),
            pl.BlockSpec(memory_space=pl.ANY),
        ),
        scratch_shapes=[
            pltpu.VMEM((m_per, n), jnp.float32),
            pltpu.VMEM((N_DEV, m_per, k_per), jnp.float32),
            pltpu.VMEM((N_DEV - 1, m_per, k_per), jnp.bfloat16),
            pltpu.VMEM((2, m_per, k_per), jnp.bfloat16),
            pltpu.VMEM((2, k_per, n), jnp.float32),
            pltpu.SemaphoreType.DMA((2 * N_DEV,)),
            pltpu.SemaphoreType.DMA((2 * (N_DEV - 1),)),
            pltpu.SemaphoreType.DMA((2 * (N_DEV - 1),)),
            pltpu.SemaphoreType.DMA((2,)),
            pltpu.SemaphoreType.DMA((2,)),
            pltpu.SemaphoreType.DMA((4,)),
        ],
        compiler_params=pltpu.CompilerParams(
            collective_id=0,
            vmem_limit_bytes=100 * 1024 * 1024,
        ),
    )(x, w_mat)
    return out


# device time: 70544 ns/iter; 1.0286x vs baseline; 1.0286x over previous
import jax
import jax.numpy as jnp
from jax import lax
from jax.experimental import pallas as pl
from jax.experimental.pallas import tpu as pltpu

N_DEV = 4


def kernel(x, w_mat):
    m_total, k_per = x.shape
    k_total, n = w_mat.shape
    m_per = m_total // N_DEV

    def body(x_hbm, w_hbm, out_hbm, acc_ref, xstage_ref, xb_ref, comm_ref,
             wstage_ref, xsems, send_sems, recv_sems, wsems, out_sems):
        my = lax.axis_index("i")
        m_h = m_per // 2

        def x_copy(row, slot, h):
            return pltpu.make_async_copy(
                x_hbm.at[pl.ds(row * m_per + h * m_h, m_h), :],
                xstage_ref.at[slot, pl.ds(h * m_h, m_h)],
                xsems.at[2 * slot + h],
            )

        xcs = []
        for d in range(1, N_DEV):
            row = lax.rem(my + d, N_DEV)
            halves = []
            for h in range(2):
                c = x_copy(row, d - 1, h)
                c.start()
                halves.append(c)
            xcs.append(halves)
        xc_own = []
        for h in range(2):
            c = x_copy(my, N_DEV - 1, h)
            c.start()
            xc_own.append(c)

        def w_copy(kb, slot):
            return pltpu.make_async_copy(
                w_hbm.at[pl.ds(kb * k_per, k_per), :],
                wstage_ref.at[slot],
                wsems.at[slot],
            )

        wc = [w_copy(my, 0)]
        wc[0].start()

        barrier = pltpu.get_barrier_semaphore()
        for d in range(1, N_DEV):
            peer = lax.rem(my + d, N_DEV)
            pl.semaphore_signal(
                barrier, inc=1,
                device_id=(peer,), device_id_type=pl.DeviceIdType.MESH,
            )
        pl.semaphore_wait(barrier, N_DEV - 1)

        rdmas = []
        for d in range(1, N_DEV):
            peer = lax.rem(my + d, N_DEV)
            halves = []
            for h in range(2):
                rs = pl.ds(h * m_h, m_h)
                xcs[d - 1][h].wait()
                xb_ref[d - 1, rs] = xstage_ref[d - 1][h * m_h:(h + 1) * m_h,
                                                      :].astype(jnp.bfloat16)
                rdma = pltpu.make_async_remote_copy(
                    src_ref=xb_ref.at[d - 1, rs],
                    dst_ref=comm_ref.at[d - 1, rs],
                    send_sem=send_sems.at[2 * (d - 1) + h],
                    recv_sem=recv_sems.at[2 * (d - 1) + h],
                    device_id=(peer,),
                    device_id_type=pl.DeviceIdType.MESH,
                )
                rdma.start()
                halves.append(rdma)
            rdmas.append(halves)

        seq = [1, 3, 2]
        wc.append(w_copy(lax.rem(my - seq[0] + N_DEV, N_DEV), 1))
        wc[1].start()

        for c in xc_own:
            c.wait()
        wc[0].wait()
        acc_ref[...] = jnp.dot(
            xstage_ref[N_DEV - 1].astype(jnp.bfloat16),
            wstage_ref[0].astype(jnp.bfloat16),
            preferred_element_type=jnp.float32,
        )

        for j in range(2):
            d = seq[j]
            slot = (j + 1) % 2
            nxt = w_copy(lax.rem(my - seq[j + 1] + N_DEV, N_DEV), j % 2)
            nxt.start()
            wc.append(nxt)
            rdmas[d - 1][0].wait_recv()
            rdmas[d - 1][1].wait_recv()
            wc[j + 1].wait()
            acc_ref[...] += jnp.dot(
                comm_ref[d - 1],
                wstage_ref[slot].astype(jnp.bfloat16),
                preferred_element_type=jnp.float32,
            )

        d = seq[2]
        rdmas[d - 1][0].wait_recv()
        rdmas[d - 1][1].wait_recv()
        wc[3].wait()
        w_last = wstage_ref[1].astype(jnp.bfloat16)
        n_chunks = 4
        m_c = m_per // n_chunks
        out_copies = []
        for c in range(n_chunks):
            rs = pl.ds(c * m_c, m_c)
            y = acc_ref[rs, :] + jnp.dot(
                comm_ref[d - 1][c * m_c:(c + 1) * m_c, :],
                w_last,
                preferred_element_type=jnp.float32,
            )
            acc_ref[rs, :] = y * jax.nn.sigmoid(y)
            oc = pltpu.make_async_copy(
                acc_ref.at[rs, :], out_hbm.at[rs, :], out_sems.at[c]
            )
            oc.start()
            out_copies.append(oc)

        for halves in rdmas:
            for r in halves:
                r.wait_send()
        for oc in out_copies:
            oc.wait()

    return pl.pallas_call(
        body,
        out_shape=jax.ShapeDtypeStruct((m_per, n), jnp.float32),
        in_specs=[
            pl.BlockSpec(memory_space=pl.ANY),
            pl.BlockSpec(memory_space=pl.ANY),
        ],
        out_specs=pl.BlockSpec(memory_space=pl.ANY),
        scratch_shapes=[
            pltpu.VMEM((m_per, n), jnp.float32),
            pltpu.VMEM((N_DEV, m_per, k_per), jnp.float32),
            pltpu.VMEM((N_DEV - 1, m_per, k_per), jnp.bfloat16),
            pltpu.VMEM((N_DEV - 1, m_per, k_per), jnp.bfloat16),
            pltpu.VMEM((2, k_per, n), jnp.float32),
            pltpu.SemaphoreType.DMA((2 * N_DEV,)),
            pltpu.SemaphoreType.DMA((2 * (N_DEV - 1),)),
            pltpu.SemaphoreType.DMA((2 * (N_DEV - 1),)),
            pltpu.SemaphoreType.DMA((2,)),
            pltpu.SemaphoreType.DMA((4,)),
        ],
        compiler_params=pltpu.CompilerParams(
            collective_id=0,
            vmem_limit_bytes=100 * 1024 * 1024,
        ),
    )(x, w_mat)


# device time: 69604 ns/iter; 1.0425x vs baseline; 1.0135x over previous
import jax
import jax.numpy as jnp
from jax import lax
from jax.experimental import pallas as pl
from jax.experimental.pallas import tpu as pltpu

N_DEV = 4


def kernel(x, w_mat):
    m_total, k_per = x.shape
    k_total, n = w_mat.shape
    m_per = m_total // N_DEV

    def body(x_hbm, w_hbm, out_hbm, acc_ref, xstage_ref, xb_ref, comm_ref,
             wstage_ref, xsems, send_sems, recv_sems, wsems, out_sems):
        my = lax.axis_index("i")

        m_h = m_per // 2

        def x_copy(row, slot, h):
            return pltpu.make_async_copy(
                x_hbm.at[pl.ds(row * m_per + h * m_h, m_h), :],
                xstage_ref.at[slot, pl.ds(h * m_h, m_h)],
                xsems.at[2 * slot + h],
            )

        xcs = []
        for d in range(1, N_DEV):
            row = lax.rem(my + d, N_DEV)
            halves = []
            for h in range(2):
                c = x_copy(row, d - 1, h)
                c.start()
                halves.append(c)
            xcs.append(halves)
        xc_own = []
        for h in range(2):
            c = x_copy(my, N_DEV - 1, h)
            c.start()
            xc_own.append(c)

        def w_copy(kb, slot):
            return pltpu.make_async_copy(
                w_hbm.at[pl.ds(kb * k_per, k_per), :],
                wstage_ref.at[slot],
                wsems.at[slot],
            )

        wc = [w_copy(my, 0)]
        wc[0].start()

        barrier = pltpu.get_barrier_semaphore()
        for d in range(1, N_DEV):
            peer = lax.rem(my + d, N_DEV)
            pl.semaphore_signal(
                barrier, inc=1,
                device_id=(peer,), device_id_type=pl.DeviceIdType.MESH,
            )
        pl.semaphore_wait(barrier, N_DEV - 1)

        rdmas = []
        for d in range(1, N_DEV):
            peer = lax.rem(my + d, N_DEV)
            halves = []
            for h in range(2):
                rs = pl.ds(h * m_h, m_h)
                xcs[d - 1][h].wait()
                xb_ref[d - 1, rs] = xstage_ref[d - 1][h * m_h:(h + 1) * m_h,
                                                      :].astype(jnp.bfloat16)
                rdma = pltpu.make_async_remote_copy(
                    src_ref=xb_ref.at[d - 1, rs],
                    dst_ref=comm_ref.at[d - 1, rs],
                    send_sem=send_sems.at[2 * (d - 1) + h],
                    recv_sem=recv_sems.at[2 * (d - 1) + h],
                    device_id=(peer,),
                    device_id_type=pl.DeviceIdType.MESH,
                )
                rdma.start()
                halves.append(rdma)
            rdmas.append(halves)

        wc.append(w_copy(lax.rem(my + N_DEV - 1, N_DEV), 1))
        wc[1].start()

        for c in xc_own:
            c.wait()
        wc[0].wait()
        acc_ref[...] = jnp.dot(
            xstage_ref[N_DEV - 1].astype(jnp.bfloat16),
            wstage_ref[0].astype(jnp.bfloat16),
            preferred_element_type=jnp.float32,
        )

        for d in range(1, N_DEV - 1):
            slot = d % 2
            kb_next = lax.rem(my - (d + 1) + N_DEV, N_DEV)
            nxt = w_copy(kb_next, (d + 1) % 2)
            nxt.start()
            wc.append(nxt)
            rdmas[d - 1][0].wait_recv()
            rdmas[d - 1][1].wait_recv()
            wc[d].wait()
            acc_ref[...] += jnp.dot(
                comm_ref[d - 1],
                wstage_ref[slot].astype(jnp.bfloat16),
                preferred_element_type=jnp.float32,
            )

        d = N_DEV - 1
        rdmas[d - 1][0].wait_recv()
        rdmas[d - 1][1].wait_recv()
        wc[d].wait()
        w_last = wstage_ref[d % 2].astype(jnp.bfloat16)
        n_chunks = 4
        m_c = m_per // n_chunks
        out_copies = []
        for c in range(n_chunks):
            rs = pl.ds(c * m_c, m_c)
            y = acc_ref[rs, :] + jnp.dot(
                comm_ref[d - 1][c * m_c:(c + 1) * m_c, :],
                w_last,
                preferred_element_type=jnp.float32,
            )
            acc_ref[rs, :] = y * jax.nn.sigmoid(y)
            oc = pltpu.make_async_copy(
                acc_ref.at[rs, :], out_hbm.at[rs, :], out_sems.at[c]
            )
            oc.start()
            out_copies.append(oc)

        for halves in rdmas:
            for r in halves:
                r.wait_send()
        for oc in out_copies:
            oc.wait()

    return pl.pallas_call(
        body,
        out_shape=jax.ShapeDtypeStruct((m_per, n), jnp.float32),
        in_specs=[
            pl.BlockSpec(memory_space=pl.ANY),
            pl.BlockSpec(memory_space=pl.ANY),
        ],
        out_specs=pl.BlockSpec(memory_space=pl.ANY),
        scratch_shapes=[
            pltpu.VMEM((m_per, n), jnp.float32),
            pltpu.VMEM((N_DEV, m_per, k_per), jnp.float32),
            pltpu.VMEM((N_DEV - 1, m_per, k_per), jnp.bfloat16),
            pltpu.VMEM((N_DEV - 1, m_per, k_per), jnp.bfloat16),
            pltpu.VMEM((2, k_per, n), jnp.float32),
            pltpu.SemaphoreType.DMA((2 * N_DEV,)),
            pltpu.SemaphoreType.DMA((2 * (N_DEV - 1),)),
            pltpu.SemaphoreType.DMA((2 * (N_DEV - 1),)),
            pltpu.SemaphoreType.DMA((2,)),
            pltpu.SemaphoreType.DMA((4,)),
        ],
        compiler_params=pltpu.CompilerParams(
            collective_id=0,
            vmem_limit_bytes=100 * 1024 * 1024,
        ),
    )(x, w_mat)
